# baseline (speedup 1.0000x reference)
import functools

import jax
import jax.numpy as jnp
from jax import lax
from jax.experimental import pallas as pl
from jax.experimental.pallas import tpu as pltpu
from jax.experimental.pallas import tpu_sc as plsc

MAXLEN = 200
EMBED_DIM = 128
BATCH = 1024

NC = 2
NS = 16
NW = NC * NS
ROWS_PER_W = BATCH * MAXLEN // NW
CHUNK = 128
N_CHUNKS = ROWS_PER_W // CHUNK
NBUF = 5
MAIN_TRIPS = N_CHUNKS // NBUF


def _body(x_hbm, tok_hbm, pos2_hbm, out_hbm, idx_v, pos_v, *bufsem):
    bufs = bufsem[:NBUF]
    gsems = bufsem[NBUF:2 * NBUF]
    wsems = bufsem[2 * NBUF:3 * NBUF]
    psems = bufsem[3 * NBUF:4 * NBUF]
    wid = lax.axis_index("s") * NC + lax.axis_index("c")

    pltpu.sync_copy(x_hbm.at[wid], idx_v)

    @pl.when(lax.axis_index("s") == 0)
    def _():
        pltpu.sync_copy(pos2_hbm, pos_v)

    plsc.subcore_barrier()

    def issue_prefill(c, k):
        off = lax.rem(c * CHUNK, MAXLEN)
        pltpu.async_copy(pos_v.at[pl.ds(off, CHUNK)], bufs[k], psems[k])

    def wait_prefill(k):
        pltpu.make_async_copy(pos_v.at[pl.ds(0, CHUNK)], bufs[k], psems[k]).wait()

    def issue_gather(c, k):
        pltpu.async_copy(tok_hbm.at[idx_v.at[c]], bufs[k], gsems[k], add=True)

    def wait_gather(k):
        pltpu.make_async_copy(tok_hbm.at[pl.ds(0, CHUNK)], bufs[k], gsems[k]).wait()

    def issue_wb(c, k):
        row = wid * ROWS_PER_W + c * CHUNK
        pltpu.async_copy(bufs[k], out_hbm.at[pl.ds(row, CHUNK)], wsems[k])

    def wait_wb(k):
        pltpu.make_async_copy(bufs[k], out_hbm.at[pl.ds(0, CHUNK)], wsems[k]).wait()

    issue_prefill(0, 0)
    wait_prefill(0)
    issue_gather(0, 0)
    issue_prefill(1, 1)

    def step(u, carry):
        for k in range(NBUF):
            c = u * NBUF + k
            kn = (k + 1) % NBUF
            kp = (k + 2) % NBUF
            wait_gather(k)
            issue_wb(c, k)

            if k == NBUF - 1:
                @pl.when(u < MAIN_TRIPS - 1)
                def _(c=c, kn=kn):
                    wait_prefill(kn)
                    issue_gather(c + 1, kn)
            else:
                wait_prefill(kn)
                issue_gather(c + 1, kn)

            if k < 3:
                @pl.when(u > 0)
                def _(kp=kp):
                    wait_wb(kp)
            else:
                wait_wb(kp)
            if k >= NBUF - 2:
                @pl.when(u < MAIN_TRIPS - 1)
                def _(c=c, kp=kp):
                    issue_prefill(c + 2, kp)
            else:
                issue_prefill(c + 2, kp)
        return carry

    lax.fori_loop(0, MAIN_TRIPS, step, 0, unroll=False)

    wait_wb((N_CHUNKS - 3) % NBUF)
    wait_wb((N_CHUNKS - 2) % NBUF)
    wait_wb((N_CHUNKS - 1) % NBUF)


@jax.jit
def _embed(x2, token_table, pos2):
    mesh = plsc.VectorSubcoreMesh(
        core_axis_name="c", subcore_axis_name="s", num_cores=NC, num_subcores=NS
    )
    run = functools.partial(
        pl.kernel,
        mesh=mesh,
        out_type=jax.ShapeDtypeStruct((BATCH * MAXLEN, EMBED_DIM), jnp.float32),
        scratch_types=[
            pltpu.VMEM((N_CHUNKS, CHUNK), jnp.int32),
            pltpu.VMEM_SHARED((2 * MAXLEN, EMBED_DIM), jnp.float32),
        ]
        + [pltpu.VMEM((CHUNK, EMBED_DIM), jnp.float32) for _ in range(NBUF)]
        + [pltpu.SemaphoreType.DMA for _ in range(3 * NBUF)],
    )(_body)
    return run(x2, token_table, pos2)


def kernel(x, token_table, pos_table):
    x2 = x.astype(jnp.int32).reshape(NW, N_CHUNKS, CHUNK)
    pos2 = jnp.concatenate([pos_table, pos_table], axis=0)
    out = _embed(x2, token_table, pos2)
    return out.reshape(BATCH, MAXLEN, EMBED_DIM)


# speedup vs baseline: 1.0952x; 1.0952x over previous
import functools

import jax
import jax.numpy as jnp
from jax import lax
from jax.experimental import pallas as pl
from jax.experimental.pallas import tpu as pltpu
from jax.experimental.pallas import tpu_sc as plsc

MAXLEN = 200
EMBED_DIM = 128
BATCH = 1024

NC = 2
NS = 16
NW = NC * NS
SEQ_PER_W = BATCH // NW
IDX_CHUNK = 100
PAIR_ROWS = 2 * MAXLEN
N_PAIRS = SEQ_PER_W // 2
CHUNKS_PER_PAIR = PAIR_ROWS // IDX_CHUNK
MAIN_TRIPS = N_PAIRS // 2


def _body(x_hbm, tok_hbm, pos_hbm, out_hbm, idx_v, pos_v, *bufsem):
    bufs = bufsem[:2]
    gsems = bufsem[2:4]
    wsems = bufsem[4:6]
    psems = bufsem[6:10]
    wid = lax.axis_index("s") * NC + lax.axis_index("c")

    pltpu.sync_copy(x_hbm.at[wid], idx_v)

    @pl.when(lax.axis_index("s") == 0)
    def _():
        pltpu.sync_copy(pos_hbm, pos_v)

    plsc.subcore_barrier()

    def issue_prefill(k):
        for h in range(2):
            pltpu.async_copy(
                pos_v, bufs[k].at[pl.ds(h * MAXLEN, MAXLEN)], psems[2 * k + h])

    def wait_prefill(k, h):
        pltpu.make_async_copy(
            pos_v, bufs[k].at[pl.ds(h * MAXLEN, MAXLEN)], psems[2 * k + h]).wait()

    def issue_gather_half(p, k, h):
        for j in range(2):
            jj = h * 2 + j
            pltpu.async_copy(
                tok_hbm.at[idx_v.at[p * CHUNKS_PER_PAIR + jj]],
                bufs[k].at[pl.ds(jj * IDX_CHUNK, IDX_CHUNK)],
                gsems[k],
                add=True,
            )

    def wait_gather(k):
        pltpu.make_async_copy(
            tok_hbm.at[pl.ds(0, PAIR_ROWS)], bufs[k], gsems[k]).wait()

    def issue_wb(p, k):
        row = wid * (SEQ_PER_W * MAXLEN) + p * PAIR_ROWS
        pltpu.async_copy(bufs[k], out_hbm.at[pl.ds(row, PAIR_ROWS)], wsems[k])

    def wait_wb(k):
        pltpu.make_async_copy(
            bufs[k], out_hbm.at[pl.ds(0, PAIR_ROWS)], wsems[k]).wait()

    def launch(p, k):
        issue_prefill(k)
        wait_prefill(k, 0)
        issue_gather_half(p, k, 0)
        wait_prefill(k, 1)
        issue_gather_half(p, k, 1)

    launch(0, 0)

    def step(u, carry):
        for k in range(2):
            p = u * 2 + k
            ko = 1 - k
            wait_gather(k)
            issue_wb(p, k)

            if k == 0:
                @pl.when(u > 0)
                def _(ko=ko):
                    wait_wb(ko)
                launch(p + 1, ko)
            else:
                wait_wb(ko)

                @pl.when(u < MAIN_TRIPS - 1)
                def _(p=p, ko=ko):
                    launch(p + 1, ko)
        return carry

    lax.fori_loop(0, MAIN_TRIPS, step, 0, unroll=False)

    wait_wb(1)


@jax.jit
def _embed(x2, token_table, pos_table):
    mesh = plsc.VectorSubcoreMesh(
        core_axis_name="c", subcore_axis_name="s", num_cores=NC, num_subcores=NS
    )
    run = functools.partial(
        pl.kernel,
        mesh=mesh,
        out_type=jax.ShapeDtypeStruct((BATCH * MAXLEN, EMBED_DIM), jnp.float32),
        scratch_types=[
            pltpu.VMEM((SEQ_PER_W * 2, IDX_CHUNK), jnp.int32),
            pltpu.VMEM_SHARED((MAXLEN, EMBED_DIM), jnp.float32),
        ]
        + [pltpu.VMEM((PAIR_ROWS, EMBED_DIM), jnp.float32) for _ in range(2)]
        + [pltpu.SemaphoreType.DMA for _ in range(8)],
    )(_body)
    return run(x2, token_table, pos_table)


def kernel(x, token_table, pos_table):
    x2 = x.astype(jnp.int32).reshape(NW, SEQ_PER_W * 2, IDX_CHUNK)
    out = _embed(x2, token_table, pos_table)
    return out.reshape(BATCH, MAXLEN, EMBED_DIM)


# speedup vs baseline: 1.1477x; 1.0479x over previous
import functools

import jax
import jax.numpy as jnp
from jax import lax
from jax.experimental import pallas as pl
from jax.experimental.pallas import tpu as pltpu
from jax.experimental.pallas import tpu_sc as plsc

MAXLEN = 200
EMBED_DIM = 128
BATCH = 1024

NC = 2
NS = 16
NW = NC * NS
ROWS_PER_W = BATCH * MAXLEN // NW
IDX_CHUNK = 128
CHUNK = 2 * IDX_CHUNK
N_CHUNKS = ROWS_PER_W // CHUNK
NBUF = 3
MAIN_TRIPS = N_CHUNKS // NBUF
POS_REP = 3


def _body(x_hbm, tok_hbm, pos3_hbm, out_hbm, idx_v, pos_v, *bufsem):
    bufs = bufsem[:NBUF]
    gsems = bufsem[NBUF:2 * NBUF]
    wsems = bufsem[2 * NBUF:3 * NBUF]
    psems = bufsem[3 * NBUF:4 * NBUF]
    wid = lax.axis_index("s") * NC + lax.axis_index("c")

    pltpu.sync_copy(x_hbm.at[wid], idx_v)

    @pl.when(lax.axis_index("s") == 0)
    def _():
        pltpu.sync_copy(pos3_hbm, pos_v)

    plsc.subcore_barrier()

    def issue_prefill(c, k):
        off = lax.rem(c * CHUNK, MAXLEN)
        pltpu.async_copy(pos_v.at[pl.ds(off, CHUNK)], bufs[k], psems[k])

    def wait_prefill(k):
        pltpu.make_async_copy(pos_v.at[pl.ds(0, CHUNK)], bufs[k], psems[k]).wait()

    def issue_gather(c, k):
        for j in range(2):
            pltpu.async_copy(
                tok_hbm.at[idx_v.at[2 * c + j]],
                bufs[k].at[pl.ds(j * IDX_CHUNK, IDX_CHUNK)],
                gsems[k],
                add=True,
            )

    def wait_gather(k):
        pltpu.make_async_copy(tok_hbm.at[pl.ds(0, CHUNK)], bufs[k], gsems[k]).wait()

    def issue_wb(c, k):
        row = wid * ROWS_PER_W + c * CHUNK
        pltpu.async_copy(bufs[k], out_hbm.at[pl.ds(row, CHUNK)], wsems[k])

    def wait_wb(k):
        pltpu.make_async_copy(bufs[k], out_hbm.at[pl.ds(0, CHUNK)], wsems[k]).wait()

    issue_prefill(0, 0)
    wait_prefill(0)
    issue_gather(0, 0)
    issue_prefill(1, 1)

    def step(u, carry):
        for k in range(NBUF):
            c = u * NBUF + k
            kn = (k + 1) % NBUF
            kp = (k + 2) % NBUF
            wait_gather(k)
            issue_wb(c, k)

            wait_prefill(kn)
            issue_gather(c + 1, kn)

            if k == 0:
                @pl.when(u > 0)
                def _(kp=kp):
                    wait_wb(kp)
            else:
                wait_wb(kp)
            if k == NBUF - 1:
                @pl.when(u < MAIN_TRIPS - 1)
                def _(c=c, kp=kp):
                    issue_prefill(c + 2, kp)
            else:
                issue_prefill(c + 2, kp)
        return carry

    lax.fori_loop(0, MAIN_TRIPS, step, 0, unroll=False)

    wait_gather(0)
    issue_wb(N_CHUNKS - 1, 0)
    wait_wb(2)
    wait_wb(0)


@jax.jit
def _embed(x2, token_table, pos3):
    mesh = plsc.VectorSubcoreMesh(
        core_axis_name="c", subcore_axis_name="s", num_cores=NC, num_subcores=NS
    )
    run = functools.partial(
        pl.kernel,
        mesh=mesh,
        out_type=jax.ShapeDtypeStruct((BATCH * MAXLEN, EMBED_DIM), jnp.float32),
        scratch_types=[
            pltpu.VMEM((2 * N_CHUNKS, IDX_CHUNK), jnp.int32),
            pltpu.VMEM_SHARED((POS_REP * MAXLEN, EMBED_DIM), jnp.float32),
        ]
        + [pltpu.VMEM((CHUNK, EMBED_DIM), jnp.float32) for _ in range(NBUF)]
        + [pltpu.SemaphoreType.DMA for _ in range(3 * NBUF)],
    )(_body)
    return run(x2, token_table, pos3)


def kernel(x, token_table, pos_table):
    x2 = x.astype(jnp.int32).reshape(NW, 2 * N_CHUNKS, IDX_CHUNK)
    pos3 = jnp.concatenate([pos_table] * POS_REP, axis=0)
    out = _embed(x2, token_table, pos3)
    return out.reshape(BATCH, MAXLEN, EMBED_DIM)


# speedup vs baseline: 1.1696x; 1.0191x over previous
import functools

import jax
import jax.numpy as jnp
from jax import lax
from jax.experimental import pallas as pl
from jax.experimental.pallas import tpu as pltpu
from jax.experimental.pallas import tpu_sc as plsc

MAXLEN = 200
EMBED_DIM = 128
BATCH = 1024

NC = 2
NS = 16
LANES = 16
NW = NC * NS
SEQ_PER_W = BATCH // NW
IDX_CHUNK = 100
N_CHUNK = MAXLEN // IDX_CHUNK
NBUF = 4
MAIN_TRIPS = SEQ_PER_W // NBUF


def _body(x_hbm, tok_hbm, pos_hbm, out_hbm, idx_v, pos_v, *bufsem):
    bufs = bufsem[:NBUF]
    gsems = bufsem[NBUF:2 * NBUF]
    wsems = bufsem[2 * NBUF:3 * NBUF]
    psems = bufsem[3 * NBUF:4 * NBUF]
    wid = lax.axis_index("s") * NC + lax.axis_index("c")

    pltpu.sync_copy(x_hbm.at[wid], idx_v)

    @pl.when(lax.axis_index("s") == 0)
    def _():
        pltpu.sync_copy(pos_hbm, pos_v)

    plsc.subcore_barrier()

    def issue_prefill(k):
        pltpu.async_copy(pos_v, bufs[k], psems[k])

    def wait_prefill(k):
        pltpu.make_async_copy(pos_v, bufs[k], psems[k]).wait()

    def issue_gather(s, k):
        for j in range(N_CHUNK):
            pltpu.async_copy(
                tok_hbm.at[idx_v.at[s * N_CHUNK + j]],
                bufs[k].at[pl.ds(j * IDX_CHUNK, IDX_CHUNK)],
                gsems[k],
                add=True,
            )

    def wait_gather(k):
        pltpu.make_async_copy(tok_hbm.at[pl.ds(0, MAXLEN)], bufs[k], gsems[k]).wait()

    def issue_wb(s, k):
        row = wid * (SEQ_PER_W * MAXLEN) + s * MAXLEN
        pltpu.async_copy(bufs[k], out_hbm.at[pl.ds(row, MAXLEN)], wsems[k])

    def wait_wb(k):
        pltpu.make_async_copy(bufs[k], out_hbm.at[pl.ds(0, MAXLEN)], wsems[k]).wait()

    issue_prefill(0)
    wait_prefill(0)
    issue_gather(0, 0)
    issue_prefill(1)

    def step(u, carry):
        for k in range(NBUF):
            s = u * NBUF + k
            kn = (k + 1) % NBUF
            kp = (k + 2) % NBUF
            wait_gather(k)
            issue_wb(s, k)

            if k == NBUF - 1:
                @pl.when(u < MAIN_TRIPS - 1)
                def _(s=s, kn=kn):
                    wait_prefill(kn)
                    issue_gather(s + 1, kn)
            else:
                wait_prefill(kn)
                issue_gather(s + 1, kn)

            if k < 2:
                @pl.when(u > 0)
                def _(kp=kp):
                    wait_wb(kp)
            else:
                wait_wb(kp)
            if k >= 2:
                @pl.when(u < MAIN_TRIPS - 1)
                def _(kp=kp):
                    issue_prefill(kp)
            else:
                issue_prefill(kp)
        return carry

    lax.fori_loop(0, MAIN_TRIPS, step, 0, unroll=False)

    wait_wb(2)
    wait_wb(3)


@jax.jit
def _embed(x2, token_table, pos_table):
    mesh = plsc.VectorSubcoreMesh(
        core_axis_name="c", subcore_axis_name="s", num_cores=NC, num_subcores=NS
    )
    run = functools.partial(
        pl.kernel,
        mesh=mesh,
        out_type=jax.ShapeDtypeStruct((BATCH * MAXLEN, EMBED_DIM), jnp.float32),
        scratch_types=[
            pltpu.VMEM((SEQ_PER_W * N_CHUNK, IDX_CHUNK), jnp.int32),
            pltpu.VMEM_SHARED((MAXLEN, EMBED_DIM), jnp.float32),
        ]
        + [pltpu.VMEM((MAXLEN, EMBED_DIM), jnp.float32) for _ in range(NBUF)]
        + [pltpu.SemaphoreType.DMA for _ in range(3 * NBUF)],
    )(_body)
    return run(x2, token_table, pos_table)


def kernel(x, token_table, pos_table):
    x2 = x.astype(jnp.int32).reshape(NW, SEQ_PER_W * N_CHUNK, IDX_CHUNK)
    out = _embed(x2, token_table, pos_table)
    return out.reshape(BATCH, MAXLEN, EMBED_DIM)
